# Initial kernel scaffold; baseline (speedup 1.0000x reference)
import jax
import jax.numpy as jnp
from jax.experimental import pallas as pl


def kernel(x, wp, bp, wc, bc, wo, bo, sim_alpha, sim_beta):
    raise NotImplementedError("write your pallas kernel here")


# speedup vs baseline: 2.7698x; 2.7698x over previous
import functools

import numpy as np
import jax
import jax.numpy as jnp
from jax.experimental import pallas as pl
from jax.experimental.pallas import tpu as pltpu

_VMEM_LIMIT = 96 * 1024 * 1024


@functools.lru_cache(maxsize=None)
def _pool_matrix_full(w, h, pw, ph, fw, fh):
    H0 = h * fh
    M = pw * ph
    P = np.zeros((fw, w * H0, fh * M), dtype=np.float32)
    for f1 in range(fw):
        for f2 in range(fh):
            for i in range(pw):
                ws, we = (i * w) // pw, -(-((i + 1) * w) // pw)
                for j in range(ph):
                    hs, he = (j * h) // ph, -(-((j + 1) * h) // ph)
                    cnt = float((we - ws) * (he - hs))
                    for iw in range(ws, we):
                        for ih in range(hs, he):
                            p = iw * H0 + f2 * h + ih
                            P[f1, p, f2 * M + (i * ph + j)] = 1.0 / cnt
    return P


def _fused_kernel(ab_ref, wcv_ref, bcv_ref, wo_ref, bo_ref, pool_ref,
                  x_ref, o_ref, *, heads, head_dim, M, h, H0):
    alpha = ab_ref[0]
    beta = ab_ref[1]
    Cd = heads * head_dim

    x = x_ref[0]
    proj = (jnp.dot(wcv_ref[...], x, preferred_element_type=jnp.float32)
            + bcv_ref[...])

    pool = pool_ref[0]
    PT, FM = pool.shape
    rows = jax.lax.broadcasted_iota(jnp.int32, (FM, PT), 0)
    pix = jax.lax.broadcasted_iota(jnp.int32, (FM, PT), 1)
    own = (rows // M) == (pix % H0) // h

    outs = []
    for e in range(heads):
        cen = proj[e * head_dim:(e + 1) * head_dim]
        val = proj[Cd + e * head_dim:Cd + (e + 1) * head_dim]

        centers = jnp.dot(cen, pool, preferred_element_type=jnp.float32)
        vcent = jnp.dot(val, pool, preferred_element_type=jnp.float32)

        xn = cen * jax.lax.rsqrt(jnp.maximum(
            jnp.sum(cen * cen, axis=0, keepdims=True), 1e-24))
        cn = centers * jax.lax.rsqrt(jnp.maximum(
            jnp.sum(centers * centers, axis=0, keepdims=True), 1e-24))
        cos = jax.lax.dot_general(cn, xn, (((0,), (0,)), ((), ())),
                                  preferred_element_type=jnp.float32)
        sim = jax.nn.sigmoid(beta + alpha * cos)

        sim = jnp.where(own, sim, -1.0)
        smax = jnp.max(sim, axis=0, keepdims=True)
        idx = jnp.min(jnp.where(sim == smax, rows, FM), axis=0, keepdims=True)
        hard = jnp.where(rows == idx, sim, 0.0)

        denom = jnp.sum(hard, axis=1, keepdims=True) + 1.0
        cu = jax.lax.dot_general(val, hard, (((1,), (1,)), ((), ())),
                                 preferred_element_type=jnp.float32)
        outs.append(jnp.dot(cu + vcent, hard / denom,
                            preferred_element_type=jnp.float32))

    patches = jnp.concatenate(outs, axis=0)
    out = (jnp.dot(wo_ref[...], patches, preferred_element_type=jnp.float32)
           + bo_ref[...])
    o_ref[0] = out.astype(o_ref.dtype)


def _cluster_attention(x, wp, bp, wc, bc, wo, bo, sim_alpha, sim_beta, *,
                       heads, head_dim, fold_w, fold_h, proposal_w,
                       proposal_h):
    B, dim, W0, H0 = x.shape
    out_dim = wo.shape[0]
    Cd = heads * head_dim
    fw, fh = (fold_w, fold_h) if (fold_w > 1 and fold_h > 1) else (1, 1)
    w, h = W0 // fw, H0 // fh
    M = proposal_w * proposal_h
    PT = w * H0
    FM = fh * M

    w_cv = jnp.concatenate([wc, wp], axis=0)
    b_cv = jnp.concatenate([bc, bp], axis=0).reshape(2 * Cd, 1)
    ab = jnp.stack([jnp.asarray(sim_alpha, jnp.float32),
                    jnp.asarray(sim_beta, jnp.float32)])
    pool = jnp.asarray(_pool_matrix_full(w, h, proposal_w, proposal_h, fw, fh))
    x_flat = x.reshape(B, dim, W0 * H0)

    out = pl.pallas_call(
        functools.partial(_fused_kernel, heads=heads, head_dim=head_dim,
                          M=M, h=h, H0=H0),
        out_shape=jax.ShapeDtypeStruct((B, out_dim, W0 * H0), jnp.float32),
        grid=(B, fw),
        in_specs=[
            pl.BlockSpec(memory_space=pltpu.MemorySpace.SMEM),
            pl.BlockSpec((2 * Cd, dim), lambda b, i: (0, 0)),
            pl.BlockSpec((2 * Cd, 1), lambda b, i: (0, 0)),
            pl.BlockSpec((out_dim, Cd), lambda b, i: (0, 0)),
            pl.BlockSpec((out_dim, 1), lambda b, i: (0, 0)),
            pl.BlockSpec((1, PT, FM), lambda b, i: (i, 0, 0)),
            pl.BlockSpec((1, dim, PT), lambda b, i: (b, 0, i)),
        ],
        out_specs=pl.BlockSpec((1, out_dim, PT), lambda b, i: (b, 0, i)),
        compiler_params=pltpu.CompilerParams(
            dimension_semantics=("parallel", "parallel"),
            vmem_limit_bytes=_VMEM_LIMIT),
    )(ab, w_cv, b_cv, wo, bo.reshape(out_dim, 1), pool, x_flat)
    return out.reshape(B, out_dim, W0, H0)


@jax.jit
def kernel(x, wp, bp, wc, bc, wo, bo, sim_alpha, sim_beta):
    return _cluster_attention(x, wp, bp, wc, bc, wo, bo, sim_alpha, sim_beta,
                              heads=4, head_dim=32, fold_w=2, fold_h=2,
                              proposal_w=2, proposal_h=2)
